# Initial kernel scaffold; baseline (speedup 1.0000x reference)
import jax
import jax.numpy as jnp
from jax.experimental import pallas as pl


def kernel(x, W_s, b_s, W_flr, b_flr, W_out, b_out):
    raise NotImplementedError("write your pallas kernel here")


# speedup vs baseline: 14.0539x; 14.0539x over previous
import functools

import jax
import jax.numpy as jnp
from jax import lax
from jax.experimental import pallas as pl
from jax.experimental.pallas import tpu as pltpu

N_NEIGHBOURS = 40
N_BITS = 16


def _bf16r(a):
    bits = lax.bitcast_convert_type(a, jnp.int32)
    r = bits + jnp.int32(0x7FFF) + ((bits >> 16) & 1)
    return lax.bitcast_convert_type(r & jnp.int32(-65536), jnp.float32)


def _dotx(a, b):
    return jnp.dot(a, b, preferred_element_type=jnp.float32,
                   precision=lax.Precision.HIGHEST)


def _gravnet_kernel(x_ref, coords_ref, feats_ref,
                    wo_x_ref, wo_mx_ref, wo_me_ref, bo_ref,
                    out_ref,
                    cT_s, cTr_s, c2r_s, fT_s,
                    *, R, V, n_prop, n_dim):
    j = pl.program_id(1)
    K = N_NEIGHBOURS
    KN = K - 1

    @pl.when(j == 0)
    def _precompute():
        coords = coords_ref[0]
        cT = coords.T
        cT_s[...] = cT
        cTr_s[...] = _bf16r(cT)
        c2r_s[...] = jnp.sum(cT * cT, axis=0, keepdims=True)
        fT_s[...] = feats_ref[0].T

    rows = pl.ds(j * R, R)
    cb = coords_ref[0, rows, :]
    cbr = _bf16r(cb)
    g = cbr[:, 0:1] * cTr_s[0:1, :]
    for dd in range(1, n_dim):
        g = g + cbr[:, dd:dd + 1] * cTr_s[dd:dd + 1, :]
    c2b = jnp.sum(cb * cb, axis=1, keepdims=True)
    draw = c2b + c2r_s[...] - 2.0 * g
    bits = lax.bitcast_convert_type(draw, jnp.int32)
    key = jnp.where(bits >= 0, bits, -(bits & jnp.int32(0x7FFFFFFF)))

    def bit_body(i, p):
        b = 30 - i
        c = p | (jnp.int32(1) << b)
        cnt = jnp.sum((key < c).astype(jnp.int32), axis=1, keepdims=True)
        return jnp.where(cnt >= K, p, c)

    p = lax.fori_loop(0, N_BITS, bit_body, jnp.zeros((R, 1), jnp.int32))
    p = p | jnp.int32((1 << (31 - N_BITS)) - 1)

    colg = lax.broadcasted_iota(jnp.int32, (R, V), 1)
    m0val = jnp.min(key, axis=1, keepdims=True)
    m0idx = jnp.min(jnp.where(key == m0val, colg, V), axis=1, keepdims=True)
    selmask = (key <= p) & (colg != m0idx)

    w = jnp.exp(-10.0 * jnp.abs(draw))
    wm = jnp.where(selmask, w, 0.0)

    mean = _dotx(wm, fT_s[...].T) * (1.0 / KN)

    negbias = jnp.where(selmask, 0.0, jnp.float32(-3.0e38))
    mx_cols = []
    for c in range(n_prop):
        f_c = fT_s[c, :].reshape(1, V)
        prod = wm * f_c + negbias
        mx_cols.append(jnp.max(prod, axis=1, keepdims=True))
    mx = jnp.concatenate(mx_cols, axis=1)

    xb = x_ref[0, rows, :]
    acc = _dotx(xb, wo_x_ref[...])
    acc += _dotx(mx, wo_mx_ref[...])
    acc += _dotx(mean, wo_me_ref[...])
    out_ref[0] = jnp.tanh(acc + bo_ref[...])


def kernel(x, W_s, b_s, W_flr, b_flr, W_out, b_out):
    B, V, F = x.shape
    n_dim = W_s.shape[1]
    n_prop = W_flr.shape[1]
    n_filt = W_out.shape[1]
    R = 256
    grid = (B, V // R)

    coords = jnp.matmul(x, W_s) + b_s
    feats = jnp.matmul(x, W_flr) + b_flr

    Wo_x = W_out[:F]
    Wo_mx = W_out[F:F + n_prop]
    Wo_me = W_out[F + n_prop:]

    body = functools.partial(_gravnet_kernel, R=R, V=V,
                             n_prop=n_prop, n_dim=n_dim)

    return pl.pallas_call(
        body,
        grid=grid,
        in_specs=[
            pl.BlockSpec((1, V, F), lambda b, j: (b, 0, 0)),
            pl.BlockSpec((1, V, n_dim), lambda b, j: (b, 0, 0)),
            pl.BlockSpec((1, V, n_prop), lambda b, j: (b, 0, 0)),
            pl.BlockSpec((F, n_filt), lambda b, j: (0, 0)),
            pl.BlockSpec((n_prop, n_filt), lambda b, j: (0, 0)),
            pl.BlockSpec((n_prop, n_filt), lambda b, j: (0, 0)),
            pl.BlockSpec((1, n_filt), lambda b, j: (0, 0)),
        ],
        out_specs=pl.BlockSpec((1, R, n_filt), lambda b, j: (b, j, 0)),
        out_shape=jax.ShapeDtypeStruct((B, V, n_filt), jnp.float32),
        scratch_shapes=[
            pltpu.VMEM((n_dim, V), jnp.float32),
            pltpu.VMEM((n_dim, V), jnp.float32),
            pltpu.VMEM((1, V), jnp.float32),
            pltpu.VMEM((n_prop, V), jnp.float32),
        ],
        compiler_params=pltpu.CompilerParams(
            dimension_semantics=("arbitrary", "arbitrary"),
        ),
    )(x, coords, feats, Wo_x, Wo_mx, Wo_me, b_out.reshape(1, n_filt))
